# baseline (device time: 394738 ns/iter reference)
import jax
import jax.numpy as jnp
from jax import lax
from jax.experimental import pallas as pl
from jax.experimental.pallas import tpu as pltpu

M = 4096
D = 4096
F_SHARD = 8192
Q = 1024

BN = 1024
BK = 512
NT = D // BN
KS = F_SHARD // BK

CW = 512
NCH = D // CW

MESH = pl.DeviceIdType.MESH


def _body(s_ref, a_ref, b_ref, out_ref, acc0, acc1, p_th, m_buf,
          r_y1, r_agx, r_agy, r_diag,
          rs_s, rs_r, agx_s, agx_r, agy_s, agy_r, fwd_s, fwd_r,
          cp_my, cp_x, cp_y, cp_d):
    n = pl.program_id(0)
    k = pl.program_id(1)
    m = pl.program_id(2)

    my_x = lax.axis_index("x")
    my_y = lax.axis_index("y")
    nbr_y = (my_x, 1 - my_y)
    nbr_x = (1 - my_x, my_y)
    my_q = 2 * my_x + my_y
    qx = 2 * (1 - my_x) + my_y
    qy = 2 * my_x + (1 - my_y)
    qd = 2 * (1 - my_x) + (1 - my_y)

    @pl.when(jnp.logical_and(n == 0, jnp.logical_and(k == 0, m == 0)))
    def _():
        barrier = pltpu.get_barrier_semaphore()
        pl.semaphore_signal(barrier, 1, device_id=nbr_y, device_id_type=MESH)
        pl.semaphore_signal(barrier, 1, device_id=nbr_x, device_id_type=MESH)
        pl.semaphore_wait(barrier, 2)

    a = a_ref[...].astype(jnp.bfloat16)
    b = b_ref[...].astype(jnp.bfloat16)
    prod = lax.dot_general(
        a, b, (((1,), (1,)), ((), ())), preferred_element_type=jnp.float32
    )

    @pl.when(m == 0)
    def _():
        @pl.when(k == 0)
        def _():
            acc0[...] = jnp.zeros_like(acc0)
        acc0[...] += prod

    @pl.when(m == 1)
    def _():
        @pl.when(k == 0)
        def _():
            acc1[...] = jnp.zeros_like(acc1)
        acc1[...] += prod

    def ocols(c):
        return pl.ds(c * CW, CW)

    def make_rs(c):
        return pltpu.make_async_remote_copy(
            src_ref=p_th.at[c % 2], dst_ref=r_y1.at[c],
            send_sem=rs_s.at[c], recv_sem=rs_r.at[c],
            device_id=nbr_y, device_id_type=MESH,
        )

    def make_agx(c):
        return pltpu.make_async_remote_copy(
            src_ref=m_buf.at[c], dst_ref=r_agx.at[c],
            send_sem=agx_s.at[c], recv_sem=agx_r.at[c],
            device_id=nbr_x, device_id_type=MESH,
        )

    def make_agy(c):
        return pltpu.make_async_remote_copy(
            src_ref=m_buf.at[c], dst_ref=r_agy.at[c],
            send_sem=agy_s.at[c], recv_sem=agy_r.at[c],
            device_id=nbr_y, device_id_type=MESH,
        )

    def make_fwd(c):
        return pltpu.make_async_remote_copy(
            src_ref=r_agy.at[c], dst_ref=r_diag.at[c],
            send_sem=fwd_s.at[c], recv_sem=fwd_r.at[c],
            device_id=nbr_x, device_id_type=MESH,
        )

    def out_copy(buf, c, quarter, sem):
        return pltpu.make_async_copy(
            buf.at[c], out_ref.at[pl.ds(quarter * Q, Q), ocols(c)], sem.at[c]
        )

    def process_add(c):
        make_rs(c).wait()
        s32 = m_buf[c].astype(jnp.float32) + r_y1[c].astype(jnp.float32)
        m_buf[c] = s32.astype(jnp.bfloat16)
        out_copy(m_buf, c, my_q, cp_my).start()
        make_agx(c).start()
        make_agy(c).start()

    def process_fwd(c):
        make_agy(c).wait()
        make_fwd(c).start()
        out_copy(r_agy, c, qy, cp_y).start()
        make_agx(c).wait()
        out_copy(r_agx, c, qx, cp_x).start()

    def at_step(i, kk, mm):
        return jnp.logical_and(n == i, jnp.logical_and(k == kk, m == mm))

    for i in range(NT):
        if i >= 2:
            for j in range(2):
                @pl.when(at_step(i, 4 + 2 * j, 0))
                def _(c=2 * (i - 2) + j):
                    process_fwd(c)

        if i >= 1:
            for j in range(2):
                @pl.when(at_step(i, 8 + 2 * j, 0))
                def _(c=2 * (i - 1) + j):
                    process_add(c)

        @pl.when(at_step(i, KS - 1, 0))
        def _(i=i):
            for j, c in enumerate((2 * i, 2 * i + 1)):
                p_th[j] = acc0[:, pl.ds(j * CW, CW)].astype(jnp.bfloat16)
                make_rs(c).start()

        @pl.when(at_step(i, KS - 1, 1))
        def _(i=i):
            for j, c in enumerate((2 * i, 2 * i + 1)):
                m_buf[c] = acc1[:, pl.ds(j * CW, CW)].astype(jnp.bfloat16)
            if i == NT - 1:
                for c in (NCH - 2, NCH - 1):
                    process_add(c)
                for c in (2 * (NT - 2), 2 * (NT - 2) + 1, NCH - 2, NCH - 1):
                    process_fwd(c)
                for c in range(NCH):
                    make_fwd(c).wait()
                    out_copy(r_diag, c, qd, cp_d).start()
                for c in range(NCH):
                    out_copy(m_buf, c, my_q, cp_my).wait()
                    out_copy(r_agx, c, qx, cp_x).wait()
                    out_copy(r_agy, c, qy, cp_y).wait()
                    out_copy(r_diag, c, qd, cp_d).wait()


def kernel(dy, W):
    my_x = lax.axis_index("x")
    my_y = lax.axis_index("y")
    sref = jnp.stack([my_x, my_y]).astype(jnp.int32)

    chunk = (NCH, Q, CW)
    grid_spec = pltpu.PrefetchScalarGridSpec(
        num_scalar_prefetch=1,
        grid=(NT, KS, 2),
        in_specs=[
            pl.BlockSpec(
                (Q, BK),
                lambda n, k, m, s: (2 * s[0] + (1 - m) * (1 - s[1]) + m * s[1], k),
            ),
            pl.BlockSpec((BN, BK), lambda n, k, m, s: (n, k)),
        ],
        out_specs=pl.BlockSpec(memory_space=pl.ANY),
        scratch_shapes=[
            pltpu.VMEM((Q, BN), jnp.float32),
            pltpu.VMEM((Q, BN), jnp.float32),
            pltpu.VMEM((2, Q, CW), jnp.bfloat16),
            pltpu.VMEM(chunk, jnp.bfloat16),
            pltpu.VMEM(chunk, jnp.bfloat16),
            pltpu.VMEM(chunk, jnp.bfloat16),
            pltpu.VMEM(chunk, jnp.bfloat16),
            pltpu.VMEM(chunk, jnp.bfloat16),
            pltpu.SemaphoreType.DMA((NCH,)),
            pltpu.SemaphoreType.DMA((NCH,)),
            pltpu.SemaphoreType.DMA((NCH,)),
            pltpu.SemaphoreType.DMA((NCH,)),
            pltpu.SemaphoreType.DMA((NCH,)),
            pltpu.SemaphoreType.DMA((NCH,)),
            pltpu.SemaphoreType.DMA((NCH,)),
            pltpu.SemaphoreType.DMA((NCH,)),
            pltpu.SemaphoreType.DMA((NCH,)),
            pltpu.SemaphoreType.DMA((NCH,)),
            pltpu.SemaphoreType.DMA((NCH,)),
            pltpu.SemaphoreType.DMA((NCH,)),
        ],
    )
    return pl.pallas_call(
        _body,
        grid_spec=grid_spec,
        out_shape=jax.ShapeDtypeStruct((M, D), jnp.bfloat16),
        compiler_params=pltpu.CompilerParams(
            collective_id=0, vmem_limit_bytes=64 * 1024 * 1024
        ),
    )(sref, dy, W)


# device time: 324873 ns/iter; 1.2151x vs baseline; 1.2151x over previous
import jax
import jax.numpy as jnp
from jax import lax
from jax.experimental import pallas as pl
from jax.experimental.pallas import tpu as pltpu

M = 4096
D = 4096
F_SHARD = 8192
Q = 1024

BN = 1024
BK = 1024
NT = D // BN
KS = F_SHARD // BK

CHUNKS = [(0, 512), (512, 512), (1024, 512), (1536, 512), (2048, 512),
          (2560, 512), (3072, 256), (3328, 256), (3584, 256), (3840, 256)]
TILE_CHUNKS = {0: [0, 1], 1: [2, 3], 2: [4, 5], 3: [6, 7, 8, 9]}
NCH = len(CHUNKS)

MESH = pl.DeviceIdType.MESH


def _body(s_ref, a_ref, b_ref, out_ref, acc0, acc1, p_th, m_buf, r_y1,
          rs_s, rs_r, agx_s, agx_r, agy_s, agy_r, fwd_s, fwd_r, cp_sems):
    n = pl.program_id(0)
    k = pl.program_id(1)
    m = pl.program_id(2)

    my_x = lax.axis_index("x")
    my_y = lax.axis_index("y")
    nbr_y = (my_x, 1 - my_y)
    nbr_x = (1 - my_x, my_y)
    my_q = 2 * my_x + my_y
    qy = 2 * my_x + (1 - my_y)

    @pl.when(jnp.logical_and(n == 0, jnp.logical_and(k == 0, m == 0)))
    def _():
        barrier = pltpu.get_barrier_semaphore()
        pl.semaphore_signal(barrier, 1, device_id=nbr_y, device_id_type=MESH)
        pl.semaphore_signal(barrier, 1, device_id=nbr_x, device_id_type=MESH)
        pl.semaphore_wait(barrier, 2)

    a = a_ref[...].astype(jnp.bfloat16)
    b = b_ref[...].astype(jnp.bfloat16)
    prod = lax.dot_general(
        a, b, (((1,), (1,)), ((), ())), preferred_element_type=jnp.float32
    )

    @pl.when(m == 0)
    def _():
        @pl.when(k == 0)
        def _():
            acc0[...] = jnp.zeros_like(acc0)
        acc0[...] += prod

    @pl.when(m == 1)
    def _():
        @pl.when(k == 0)
        def _():
            acc1[...] = jnp.zeros_like(acc1)
        acc1[...] += prod

    def cols(c):
        off, w = CHUNKS[c]
        return pl.ds(off, w)

    def make_rs(c):
        return pltpu.make_async_remote_copy(
            src_ref=p_th.at[:, cols(c)], dst_ref=r_y1.at[:, cols(c)],
            send_sem=rs_s.at[c], recv_sem=rs_r.at[c],
            device_id=nbr_y, device_id_type=MESH,
        )

    def make_agx(c):
        return pltpu.make_async_remote_copy(
            src_ref=m_buf.at[:, cols(c)],
            dst_ref=out_ref.at[pl.ds(my_q * Q, Q), cols(c)],
            send_sem=agx_s.at[c], recv_sem=agx_r.at[c],
            device_id=nbr_x, device_id_type=MESH,
        )

    def make_agy(c):
        return pltpu.make_async_remote_copy(
            src_ref=m_buf.at[:, cols(c)],
            dst_ref=out_ref.at[pl.ds(my_q * Q, Q), cols(c)],
            send_sem=agy_s.at[c], recv_sem=agy_r.at[c],
            device_id=nbr_y, device_id_type=MESH,
        )

    def make_fwd(c):
        return pltpu.make_async_remote_copy(
            src_ref=out_ref.at[pl.ds(qy * Q, Q), cols(c)],
            dst_ref=out_ref.at[pl.ds(qy * Q, Q), cols(c)],
            send_sem=fwd_s.at[c], recv_sem=fwd_r.at[c],
            device_id=nbr_x, device_id_type=MESH,
        )

    def make_out_copy(c):
        return pltpu.make_async_copy(
            m_buf.at[:, cols(c)],
            out_ref.at[pl.ds(my_q * Q, Q), cols(c)],
            cp_sems.at[c],
        )

    def process_add(c):
        make_rs(c).wait()
        s32 = (m_buf[:, cols(c)].astype(jnp.float32)
               + r_y1[:, cols(c)].astype(jnp.float32))
        m_buf[:, cols(c)] = s32.astype(jnp.bfloat16)
        make_out_copy(c).start()
        make_agx(c).start()
        make_agy(c).start()

    def process_fwd(c):
        make_agy(c).wait()
        make_fwd(c).start()

    def at_step(i, kk, mm):
        return jnp.logical_and(n == i, jnp.logical_and(k == kk, m == mm))

    for i in range(NT):
        if i >= 2:
            for j, c in enumerate(TILE_CHUNKS[i - 2]):
                @pl.when(at_step(i, KS - 6 + j, 0))
                def _(c=c):
                    process_fwd(c)

        if i >= 1:
            for j, c in enumerate(TILE_CHUNKS[i - 1]):
                @pl.when(at_step(i, KS - 4 + j, 0))
                def _(c=c):
                    process_add(c)

        @pl.when(at_step(i, KS - 1, 0))
        def _(i=i):
            p_th[:, pl.ds(i * BN, BN)] = acc0[...].astype(jnp.bfloat16)
            for c in TILE_CHUNKS[i]:
                make_rs(c).start()

        @pl.when(at_step(i, KS - 1, 1))
        def _(i=i):
            m_buf[:, pl.ds(i * BN, BN)] = acc1[...].astype(jnp.bfloat16)
            if i == NT - 1:
                for c in TILE_CHUNKS[NT - 1]:
                    process_add(c)
                for c in TILE_CHUNKS[NT - 2] + TILE_CHUNKS[NT - 1]:
                    process_fwd(c)
                for c in range(NCH):
                    make_fwd(c).wait()
                for c in range(NCH):
                    make_agx(c).wait()
                for c in range(NCH):
                    make_out_copy(c).wait()


def kernel(dy, W):
    my_x = lax.axis_index("x")
    my_y = lax.axis_index("y")
    sref = jnp.stack([my_x, my_y]).astype(jnp.int32)

    grid_spec = pltpu.PrefetchScalarGridSpec(
        num_scalar_prefetch=1,
        grid=(NT, KS, 2),
        in_specs=[
            pl.BlockSpec(
                (Q, BK),
                lambda n, k, m, s: (2 * s[0] + (1 - m) * (1 - s[1]) + m * s[1], k),
            ),
            pl.BlockSpec((BN, BK), lambda n, k, m, s: (n, k)),
        ],
        out_specs=pl.BlockSpec(memory_space=pl.ANY),
        scratch_shapes=[
            pltpu.VMEM((Q, BN), jnp.float32),
            pltpu.VMEM((Q, BN), jnp.float32),
            pltpu.VMEM((Q, D), jnp.bfloat16),
            pltpu.VMEM((Q, D), jnp.bfloat16),
            pltpu.VMEM((Q, D), jnp.bfloat16),
            pltpu.SemaphoreType.DMA((NCH,)),
            pltpu.SemaphoreType.DMA((NCH,)),
            pltpu.SemaphoreType.DMA((NCH,)),
            pltpu.SemaphoreType.DMA((NCH,)),
            pltpu.SemaphoreType.DMA((NCH,)),
            pltpu.SemaphoreType.DMA((NCH,)),
            pltpu.SemaphoreType.DMA((NCH,)),
            pltpu.SemaphoreType.DMA((NCH,)),
            pltpu.SemaphoreType.DMA((NCH,)),
        ],
    )
    return pl.pallas_call(
        _body,
        grid_spec=grid_spec,
        out_shape=jax.ShapeDtypeStruct((M, D), jnp.bfloat16),
        compiler_params=pltpu.CompilerParams(
            collective_id=0, vmem_limit_bytes=64 * 1024 * 1024
        ),
    )(sref, dy, W)


# device time: 324784 ns/iter; 1.2154x vs baseline; 1.0003x over previous
import jax
import jax.numpy as jnp
from jax import lax
from jax.experimental import pallas as pl
from jax.experimental.pallas import tpu as pltpu

M = 4096
D = 4096
F_SHARD = 8192
Q = 1024

BN = 1024
BK = 1024
NT = D // BN
KS = F_SHARD // BK

CHUNKS = [(0, 512), (512, 512), (1024, 512), (1536, 512), (2048, 512),
          (2560, 512), (3072, 256), (3328, 256), (3584, 256), (3840, 256)]
TILE_CHUNKS = {0: [0, 1], 1: [2, 3], 2: [4, 5], 3: [6, 7, 8, 9]}
NCH = len(CHUNKS)

MESH = pl.DeviceIdType.MESH


def _body(s_ref, a_ref, b_ref, out_ref, acc0, acc1, p_th, m_buf, r_y1,
          rs_s, rs_r, agx_s, agx_r, agy_s, agy_r, fwd_s, fwd_r, cp_sems):
    n = pl.program_id(0)
    k = pl.program_id(1)
    m = pl.program_id(2)

    my_x = lax.axis_index("x")
    my_y = lax.axis_index("y")
    nbr_y = (my_x, 1 - my_y)
    nbr_x = (1 - my_x, my_y)
    my_q = 2 * my_x + my_y
    qy = 2 * my_x + (1 - my_y)

    @pl.when(jnp.logical_and(n == 0, jnp.logical_and(k == 0, m == 0)))
    def _():
        barrier = pltpu.get_barrier_semaphore()
        pl.semaphore_signal(barrier, 1, device_id=nbr_y, device_id_type=MESH)
        pl.semaphore_signal(barrier, 1, device_id=nbr_x, device_id_type=MESH)
        pl.semaphore_wait(barrier, 2)

    a = a_ref[...].astype(jnp.bfloat16)
    b = b_ref[...].astype(jnp.bfloat16)
    prod = lax.dot_general(
        a, b, (((1,), (1,)), ((), ())), preferred_element_type=jnp.float32
    )

    @pl.when(m == 0)
    def _():
        @pl.when(k == 0)
        def _():
            acc0[...] = jnp.zeros_like(acc0)
        acc0[...] += prod

    @pl.when(m == 1)
    def _():
        @pl.when(k == 0)
        def _():
            acc1[...] = jnp.zeros_like(acc1)
        acc1[...] += prod

    def cols(c):
        off, w = CHUNKS[c]
        return pl.ds(off, w)

    def make_rs(c):
        return pltpu.make_async_remote_copy(
            src_ref=p_th.at[:, cols(c)], dst_ref=r_y1.at[:, cols(c)],
            send_sem=rs_s.at[c], recv_sem=rs_r.at[c],
            device_id=nbr_y, device_id_type=MESH,
        )

    def make_agx(c):
        return pltpu.make_async_remote_copy(
            src_ref=m_buf.at[:, cols(c)],
            dst_ref=out_ref.at[pl.ds(my_q * Q, Q), cols(c)],
            send_sem=agx_s.at[c], recv_sem=agx_r.at[c],
            device_id=nbr_x, device_id_type=MESH,
        )

    def make_agy(c):
        return pltpu.make_async_remote_copy(
            src_ref=m_buf.at[:, cols(c)],
            dst_ref=out_ref.at[pl.ds(my_q * Q, Q), cols(c)],
            send_sem=agy_s.at[c], recv_sem=agy_r.at[c],
            device_id=nbr_y, device_id_type=MESH,
        )

    def make_fwd(c):
        return pltpu.make_async_remote_copy(
            src_ref=out_ref.at[pl.ds(qy * Q, Q), cols(c)],
            dst_ref=out_ref.at[pl.ds(qy * Q, Q), cols(c)],
            send_sem=fwd_s.at[c], recv_sem=fwd_r.at[c],
            device_id=nbr_x, device_id_type=MESH,
        )

    def make_out_copy(c):
        return pltpu.make_async_copy(
            m_buf.at[:, cols(c)],
            out_ref.at[pl.ds(my_q * Q, Q), cols(c)],
            cp_sems.at[c],
        )

    def process_add(c):
        make_rs(c).wait()
        s32 = (m_buf[:, cols(c)].astype(jnp.float32)
               + r_y1[:, cols(c)].astype(jnp.float32))
        m_buf[:, cols(c)] = s32.astype(jnp.bfloat16)
        make_out_copy(c).start()
        make_agx(c).start()
        make_agy(c).start()

    def process_fwd(c):
        make_agy(c).wait()
        make_fwd(c).start()

    def at_step(i, kk, mm):
        return jnp.logical_and(n == i, jnp.logical_and(k == kk, m == mm))

    for i in range(NT):
        if i >= 2:
            for j, c in enumerate(TILE_CHUNKS[i - 2]):
                @pl.when(at_step(i, KS - 6 + j, 0))
                def _(c=c):
                    process_fwd(c)

        if i >= 1:
            for j, c in enumerate(TILE_CHUNKS[i - 1]):
                @pl.when(at_step(i, KS - 4 + j, 0))
                def _(c=c):
                    process_add(c)

        @pl.when(at_step(i, KS - 1, 0))
        def _(i=i):
            p_th[:, pl.ds(i * BN, BN)] = acc0[...].astype(jnp.bfloat16)
            for c in TILE_CHUNKS[i]:
                make_rs(c).start()

        @pl.when(at_step(i, KS - 1, 1))
        def _(i=i):
            m_buf[:, pl.ds(i * BN, BN)] = acc1[...].astype(jnp.bfloat16)
            if i == NT - 1:
                for c in TILE_CHUNKS[NT - 2]:
                    process_fwd(c)
                for c in TILE_CHUNKS[NT - 1]:
                    process_add(c)
                for c in TILE_CHUNKS[NT - 1]:
                    process_fwd(c)
                for c in range(NCH):
                    make_fwd(c).wait()
                for c in range(NCH):
                    make_agx(c).wait()
                for c in range(NCH):
                    make_out_copy(c).wait()


def kernel(dy, W):
    my_x = lax.axis_index("x")
    my_y = lax.axis_index("y")
    sref = jnp.stack([my_x, my_y]).astype(jnp.int32)

    grid_spec = pltpu.PrefetchScalarGridSpec(
        num_scalar_prefetch=1,
        grid=(NT, KS, 2),
        in_specs=[
            pl.BlockSpec(
                (Q, BK),
                lambda n, k, m, s: (2 * s[0] + (1 - m) * (1 - s[1]) + m * s[1], k),
            ),
            pl.BlockSpec((BN, BK), lambda n, k, m, s: (n, k)),
        ],
        out_specs=pl.BlockSpec(memory_space=pl.ANY),
        scratch_shapes=[
            pltpu.VMEM((Q, BN), jnp.float32),
            pltpu.VMEM((Q, BN), jnp.float32),
            pltpu.VMEM((Q, D), jnp.bfloat16),
            pltpu.VMEM((Q, D), jnp.bfloat16),
            pltpu.VMEM((Q, D), jnp.bfloat16),
            pltpu.SemaphoreType.DMA((NCH,)),
            pltpu.SemaphoreType.DMA((NCH,)),
            pltpu.SemaphoreType.DMA((NCH,)),
            pltpu.SemaphoreType.DMA((NCH,)),
            pltpu.SemaphoreType.DMA((NCH,)),
            pltpu.SemaphoreType.DMA((NCH,)),
            pltpu.SemaphoreType.DMA((NCH,)),
            pltpu.SemaphoreType.DMA((NCH,)),
            pltpu.SemaphoreType.DMA((NCH,)),
        ],
    )
    return pl.pallas_call(
        _body,
        grid_spec=grid_spec,
        out_shape=jax.ShapeDtypeStruct((M, D), jnp.bfloat16),
        compiler_params=pltpu.CompilerParams(
            collective_id=0, vmem_limit_bytes=64 * 1024 * 1024
        ),
    )(sref, dy, W)
